# baseline (speedup 1.0000x reference)
import functools

import jax
import jax.numpy as jnp
from jax import lax
from jax.experimental import pallas as pl
from jax.experimental.pallas import tpu as pltpu
from jax.experimental.pallas import tpu_sc as plsc

_B = 4
_C = 64
_HW = 1024
_N = _B * _HW
_K = 512
_BETA = 0.25
_NCAND = 3


def _grouped_tree_sum(d2):
    acc = None
    for g in range(8):
        r = [d2[8 * g + j : 8 * g + j + 1, :] for j in range(8)]
        t0 = r[0] + r[4]
        t1 = r[2] + r[6]
        t2 = r[1] + r[5]
        t3 = r[3] + r[7]
        gs = (t0 + t1) + (t2 + t3)
        acc = gs if acc is None else acc + gs
    return acc


def _vq_body(x_ref, cb_ref, idx_ref, loss_ref):
    x4 = x_ref[...]
    x = jnp.concatenate([x4[0], x4[1], x4[2], x4[3]], axis=1)
    cb = cb_ref[...]

    cb2 = jnp.sum(cb * cb, axis=1, keepdims=True)

    cb_hi = cb.astype(jnp.bfloat16)
    res = cb - cb_hi.astype(jnp.float32)
    cb_mid = res.astype(jnp.bfloat16)
    cb_lo = (res - cb_mid.astype(jnp.float32)).astype(jnp.bfloat16)

    xs = x * (-(2.0 ** 21))
    xh = xs.astype(jnp.bfloat16)
    xl = (xs - xh.astype(jnp.float32)).astype(jnp.bfloat16)
    s_hh = jax.lax.dot_general(
        cb_hi, xh, (((1,), (0,)), ((), ())),
        preferred_element_type=jnp.float32)
    c_cat = jnp.concatenate([cb_hi, cb_mid], axis=1)
    x_cat = jnp.concatenate([xl, xh], axis=0)
    s_x = jax.lax.dot_general(
        c_cat, x_cat, (((1,), (0,)), ((), ())),
        preferred_element_type=jnp.float32)
    s2 = (cb2 * (2.0 ** 20)) + (s_hh + s_x)

    kio = jax.lax.broadcasted_iota(jnp.int32, (_K, _N), 0)
    key = (s2.astype(jnp.int32) << 9) | kio

    g_cat = jnp.concatenate([cb_hi, cb_mid, cb_lo], axis=1)

    def _gather(onehot):
        p = jax.lax.dot_general(
            g_cat, onehot, (((0,), (0,)), ((), ())),
            preferred_element_type=jnp.float32)
        return (p[0:64] + p[64:128]) + p[128:192]

    best_d = None
    best_i = None
    best_g = None
    for t in range(_NCAND):
        mk = jnp.min(key, axis=0, keepdims=True)
        i_t = mk & 511
        eq = key == mk
        if t + 1 < _NCAND:
            key = jnp.where(eq, jnp.iinfo(jnp.int32).max, key)
        g_t = _gather(eq.astype(jnp.bfloat16))
        diff = x - g_t
        r_t = _grouped_tree_sum(diff * diff)
        if t == 0:
            best_d, best_i, best_g = r_t, i_t, g_t
        else:
            lt = (r_t < best_d) | ((r_t == best_d) & (i_t < best_i))
            best_d = jnp.where(lt, r_t, best_d)
            best_i = jnp.where(lt, i_t, best_i)
            best_g = jnp.where(lt, g_t, best_g)

    for b in range(_B):
        sl = slice(b * _HW, (b + 1) * _HW)
        idx_ref[b] = best_i[:, sl]

    loss_ref[...] = (jnp.sum(best_d, axis=1, keepdims=True)
                     * ((1.0 + _BETA) / (_N * _C)))


def kernel(inputs, codebook):
    shape = inputs.shape
    x = inputs.reshape(_B, _C, _HW)
    idx3, loss = pl.pallas_call(
        _vq_body,
        out_shape=[
            jax.ShapeDtypeStruct((_B, 1, _HW), jnp.int32),
            jax.ShapeDtypeStruct((1, 1), jnp.float32),
        ],
    )(x, codebook)
    cb_pad = jnp.concatenate(
        [codebook, jnp.zeros((_K, 128 - _C), jnp.float32)], axis=1)
    rows = _sc_gather(cb_pad, idx3.reshape(_N))[:, :_C]
    st = jnp.transpose(rows.reshape(_B, _HW, _C), (0, 2, 1))
    return (st.reshape(shape), idx3.reshape((_B,) + shape[2:]),
            loss.reshape(()))


def _sc_gather(codebook, idx_flat):
    info = plsc.get_sparse_core_info()
    nw = info.num_cores * info.num_subcores
    b_per_w = _N // nw
    mesh = plsc.VectorSubcoreMesh(core_axis_name="c", subcore_axis_name="s")

    @functools.partial(
        pl.kernel, mesh=mesh,
        out_type=jax.ShapeDtypeStruct((_N, 128), jnp.float32),
        scratch_types=[
            pltpu.VMEM((b_per_w,), jnp.int32),
            pltpu.VMEM((b_per_w, 128), jnp.float32),
            pltpu.SemaphoreType.DMA,
        ],
    )
    def k(table_hbm, idx_hbm, out_hbm, idx_v, rows_v, sem):
        wid = lax.axis_index("s") * info.num_cores + lax.axis_index("c")
        base = wid * b_per_w
        pltpu.sync_copy(idx_hbm.at[pl.ds(base, b_per_w)], idx_v)
        pltpu.async_copy(table_hbm.at[idx_v], rows_v, sem).wait()
        pltpu.sync_copy(rows_v, out_hbm.at[pl.ds(base, b_per_w)])

    return k(codebook, idx_flat)


# speedup vs baseline: 1.8760x; 1.8760x over previous
import jax
import jax.numpy as jnp
from jax.experimental import pallas as pl

_B = 4
_C = 64
_HW = 1024
_N = _B * _HW
_K = 512
_BETA = 0.25
_NCAND = 3


def _grouped_tree_sum(d2):
    acc = None
    for g in range(8):
        r = [d2[8 * g + j : 8 * g + j + 1, :] for j in range(8)]
        t0 = r[0] + r[4]
        t1 = r[2] + r[6]
        t2 = r[1] + r[5]
        t3 = r[3] + r[7]
        gs = (t0 + t1) + (t2 + t3)
        acc = gs if acc is None else acc + gs
    return acc


def _vq_body(x_ref, cb_ref, st_ref, idx_ref, loss_ref):
    x4 = x_ref[...]
    x = jnp.concatenate([x4[0], x4[1], x4[2], x4[3]], axis=1)
    cb = cb_ref[...]

    cb2 = jnp.sum(cb * cb, axis=1, keepdims=True)

    cb_hi = cb.astype(jnp.bfloat16)
    res = cb - cb_hi.astype(jnp.float32)
    cb_mid = res.astype(jnp.bfloat16)
    cb_lo = (res - cb_mid.astype(jnp.float32)).astype(jnp.bfloat16)

    xs = x * (-(2.0 ** 21))
    xh = xs.astype(jnp.bfloat16)
    xl = (xs - xh.astype(jnp.float32)).astype(jnp.bfloat16)
    s_hh = jax.lax.dot_general(
        cb_hi, xh, (((1,), (0,)), ((), ())),
        preferred_element_type=jnp.float32)
    c_cat = jnp.concatenate([cb_hi, cb_mid], axis=1)
    x_cat = jnp.concatenate([xl, xh], axis=0)
    s_x = jax.lax.dot_general(
        c_cat, x_cat, (((1,), (0,)), ((), ())),
        preferred_element_type=jnp.float32)
    s2 = (cb2 * (2.0 ** 20)) + (s_hh + s_x)

    kio = jax.lax.broadcasted_iota(jnp.int32, (_K, _N), 0)
    key = (s2.astype(jnp.int32) << 9) | kio

    g_cat = jnp.concatenate([cb_hi, cb_mid, cb_lo], axis=1)

    def _gather(onehot):
        p = jax.lax.dot_general(
            g_cat, onehot, (((0,), (0,)), ((), ())),
            preferred_element_type=jnp.float32)
        return (p[0:64] + p[64:128]) + p[128:192]

    best_d = None
    best_i = None
    best_g = None
    for t in range(_NCAND):
        mk = jnp.min(key, axis=0, keepdims=True)
        i_t = mk & 511
        eq = key == mk
        if t + 1 < _NCAND:
            key = jnp.where(eq, jnp.iinfo(jnp.int32).max, key)
        g_t = _gather(eq.astype(jnp.bfloat16))
        diff = x - g_t
        r_t = _grouped_tree_sum(diff * diff)
        if t == 0:
            best_d, best_i, best_g = r_t, i_t, g_t
        else:
            lt = (r_t < best_d) | ((r_t == best_d) & (i_t < best_i))
            best_d = jnp.where(lt, r_t, best_d)
            best_i = jnp.where(lt, i_t, best_i)
            best_g = jnp.where(lt, g_t, best_g)

    for b in range(_B):
        sl = slice(b * _HW, (b + 1) * _HW)
        st_ref[b] = best_g[:, sl]
        idx_ref[b] = best_i[:, sl]

    loss_ref[...] = (jnp.sum(best_d, axis=1, keepdims=True)
                     * ((1.0 + _BETA) / (_N * _C)))


def kernel(inputs, codebook):
    shape = inputs.shape
    x = inputs.reshape(_B, _C, _HW)
    st, idx3, loss = pl.pallas_call(
        _vq_body,
        out_shape=[
            jax.ShapeDtypeStruct((_B, _C, _HW), jnp.float32),
            jax.ShapeDtypeStruct((_B, 1, _HW), jnp.int32),
            jax.ShapeDtypeStruct((1, 1), jnp.float32),
        ],
    )(x, codebook)
    return (st.reshape(shape), idx3.reshape((_B,) + shape[2:]),
            loss.reshape(()))
